# Initial kernel scaffold; baseline (speedup 1.0000x reference)
import jax
import jax.numpy as jnp
from jax.experimental import pallas as pl


def kernel(x, orig_weight, aw1, aw2):
    raise NotImplementedError("write your pallas kernel here")


# speedup vs baseline: 2.9400x; 2.9400x over previous
import functools

import jax
import jax.numpy as jnp
from jax import lax
from jax.experimental import pallas as pl
from jax.experimental.pallas import tpu as pltpu
from jax.experimental.pallas import tpu_sc as plsc

_NC = 2
_NS = 16
_NW = _NC * _NS
_CH = 128


def _fuse_table(orig, aw1, aw2):
    v, d = orig.shape
    r = aw1.shape[1]
    bv = 4000
    assert v % bv == 0

    def body(o_ref, a1_ref, a2_ref, w_ref):
        w_ref[...] = o_ref[...] + jnp.dot(
            a1_ref[...], a2_ref[...], preferred_element_type=jnp.float32
        )

    return pl.pallas_call(
        body,
        grid=(v // bv,),
        in_specs=[
            pl.BlockSpec((bv, d), lambda i: (i, 0)),
            pl.BlockSpec((bv, r), lambda i: (i, 0)),
            pl.BlockSpec((r, d), lambda i: (0, 0)),
        ],
        out_specs=pl.BlockSpec((bv, d), lambda i: (i, 0)),
        out_shape=jax.ShapeDtypeStruct((v, d), jnp.float32),
    )(orig, aw1, aw2)


def _sc_gather(weight, x2d):
    n_chunks = x2d.shape[0]
    d = weight.shape[1]
    per_w = n_chunks // _NW
    assert per_w * _NW == n_chunks
    n = n_chunks * _CH
    mesh = plsc.VectorSubcoreMesh(
        core_axis_name="c", subcore_axis_name="s",
        num_cores=_NC, num_subcores=_NS,
    )

    @functools.partial(
        pl.kernel,
        out_type=jax.ShapeDtypeStruct((n, d), jnp.float32),
        mesh=mesh,
        compiler_params=pltpu.CompilerParams(use_tc_tiling_on_sc=False),
        scratch_types=[
            pltpu.VMEM((per_w, _CH), jnp.int32),
            pltpu.VMEM((_CH, d), jnp.float32),
            pltpu.SemaphoreType.DMA,
        ],
    )
    def k(w_hbm, x_hbm, out_hbm, idx_v, rows_v, sem):
        wid = lax.axis_index("s") * _NC + lax.axis_index("c")
        base = wid * per_w
        pltpu.sync_copy(x_hbm.at[pl.ds(base, per_w)], idx_v)

        def body(j, carry):
            pltpu.async_copy(w_hbm.at[idx_v.at[j]], rows_v, sem).wait()
            pltpu.sync_copy(rows_v, out_hbm.at[pl.ds((base + j) * _CH, _CH)])
            return carry

        lax.fori_loop(0, per_w, body, 0)

    return k(weight, x2d)


def kernel(x, orig_weight, aw1, aw2):
    b, l = x.shape
    d = orig_weight.shape[1]
    weight = _fuse_table(orig_weight, aw1, aw2)
    x2d = x.reshape(-1, _CH)
    out = _sc_gather(weight, x2d)
    return out.reshape(b, l, d)


# speedup vs baseline: 3.0623x; 1.0416x over previous
import functools

import jax
import jax.numpy as jnp
from jax import lax
from jax.experimental import pallas as pl
from jax.experimental.pallas import tpu as pltpu
from jax.experimental.pallas import tpu_sc as plsc

_NC = 2
_NS = 16
_NW = _NC * _NS
_BC = 4


def _fuse_table(orig, aw1, aw2):
    v, d = orig.shape
    r = aw1.shape[1]
    bv = 4000
    assert v % bv == 0

    def body(o_ref, a1_ref, a2_ref, w_ref):
        w_ref[...] = o_ref[...] + jnp.dot(
            a1_ref[...], a2_ref[...], preferred_element_type=jnp.float32
        )

    return pl.pallas_call(
        body,
        grid=(v // bv,),
        in_specs=[
            pl.BlockSpec((bv, d), lambda i: (i, 0)),
            pl.BlockSpec((bv, r), lambda i: (i, 0)),
            pl.BlockSpec((r, d), lambda i: (0, 0)),
        ],
        out_specs=pl.BlockSpec((bv, d), lambda i: (i, 0)),
        out_shape=jax.ShapeDtypeStruct((v, d), jnp.float32),
    )(orig, aw1, aw2)


def _sc_gather(weight, x_flat, b, l):
    d = weight.shape[1]
    n = x_flat.shape[0]
    per_w = b // _NW
    n_ch = per_w // _BC
    ch_idx = _BC * l
    assert per_w * _NW == b and n_ch * _BC == per_w and n == b * l
    mesh = plsc.VectorSubcoreMesh(
        core_axis_name="c", subcore_axis_name="s",
        num_cores=_NC, num_subcores=_NS,
    )

    @functools.partial(
        pl.kernel,
        out_type=jax.ShapeDtypeStruct((b, l, d), jnp.float32),
        mesh=mesh,
        compiler_params=pltpu.CompilerParams(use_tc_tiling_on_sc=False),
        scratch_types=[
            pltpu.VMEM((per_w * l,), jnp.int32),
            pltpu.VMEM((2, ch_idx, d), jnp.float32),
            pltpu.SemaphoreType.DMA((2,)),
        ],
    )
    def k(w_hbm, x_hbm, out_hbm, idx_v, rows_v, sems):
        wid = lax.axis_index("s") * _NC + lax.axis_index("c")
        base_i = wid * (per_w * l)
        base_b = wid * per_w
        pltpu.sync_copy(x_hbm.at[pl.ds(base_i, per_w * l)], idx_v)

        def gather(c, slot):
            return pltpu.make_async_copy(
                w_hbm.at[idx_v.at[pl.ds(c * ch_idx, ch_idx)]],
                rows_v.at[slot],
                sems.at[slot],
            )

        gather(0, 0).start()

        def body(c, carry):
            slot = lax.rem(c, 2)
            nslot = 1 - slot

            @pl.when(c + 1 < n_ch)
            def _():
                gather(c + 1, nslot).start()

            gather(c, slot).wait()
            for i in range(_BC):
                pltpu.sync_copy(
                    rows_v.at[slot, pl.ds(i * l, l)],
                    out_hbm.at[base_b + c * _BC + i],
                )
            return carry

        lax.fori_loop(0, n_ch, body, 0)

    return k(weight, x_flat)


def kernel(x, orig_weight, aw1, aw2):
    b, l = x.shape
    d = orig_weight.shape[1]
    weight = _fuse_table(orig_weight, aw1, aw2)
    return _sc_gather(weight, x.reshape(-1), b, l)


# speedup vs baseline: 3.3283x; 1.0869x over previous
import functools

import jax
import jax.numpy as jnp
from jax import lax
from jax.experimental import pallas as pl
from jax.experimental.pallas import tpu as pltpu
from jax.experimental.pallas import tpu_sc as plsc

_NC = 2
_NS = 16
_NW = _NC * _NS
_BC = 4


def _fuse_table(orig, aw1, aw2):
    v, d = orig.shape
    r = aw1.shape[1]
    bv = 10000
    assert v % bv == 0

    def body(o_ref, a1_ref, a2_ref, w_ref):
        w_ref[:, 0:d] = o_ref[...] + jnp.dot(
            a1_ref[...], a2_ref[...], preferred_element_type=jnp.float32
        )

    return pl.pallas_call(
        body,
        grid=(v // bv,),
        in_specs=[
            pl.BlockSpec((bv, d), lambda i: (i, 0)),
            pl.BlockSpec((bv, r), lambda i: (i, 0)),
            pl.BlockSpec((r, d), lambda i: (0, 0)),
        ],
        out_specs=pl.BlockSpec((bv, 128), lambda i: (i, 0)),
        out_shape=jax.ShapeDtypeStruct((v, 128), jnp.float32),
    )(orig, aw1, aw2)


def _sc_gather(weight2, x_flat, b, l, d):
    n = x_flat.shape[0]
    per_w = b // _NW
    n_ch = per_w // _BC
    ch_idx = _BC * l
    n_i = per_w * l
    assert per_w * _NW == b and n_ch * _BC == per_w and n == b * l
    mesh = plsc.VectorSubcoreMesh(
        core_axis_name="c", subcore_axis_name="s",
        num_cores=_NC, num_subcores=_NS,
    )

    @functools.partial(
        pl.kernel,
        out_type=jax.ShapeDtypeStruct((b, l, d), jnp.float32),
        mesh=mesh,
        compiler_params=pltpu.CompilerParams(use_tc_tiling_on_sc=False),
        scratch_types=[
            pltpu.VMEM((n_i,), jnp.int32),
            pltpu.VMEM((2, ch_idx, d), jnp.float32),
            pltpu.SemaphoreType.DMA((2,)),
        ],
    )
    def k(w_hbm, x_hbm, out_hbm, idx_v, rows_v, sems):
        wid = lax.axis_index("s") * _NC + lax.axis_index("c")
        base_i = wid * n_i
        base_b = wid * per_w
        pltpu.sync_copy(x_hbm.at[pl.ds(base_i, n_i)], idx_v)

        def dbl(i, carry):
            sl = pl.ds(i * 16, 16)
            idx_v[sl] = idx_v[sl] + idx_v[sl]
            return carry

        lax.fori_loop(0, n_i // 16, dbl, 0)

        def gather(c, slot):
            return pltpu.make_async_copy(
                w_hbm.at[idx_v.at[pl.ds(c * ch_idx, ch_idx)]],
                rows_v.at[slot],
                sems.at[slot],
            )

        gather(0, 0).start()

        def body(c, carry):
            slot = lax.rem(c, 2)

            @pl.when(c + 1 < n_ch)
            def _():
                gather(c + 1, 1 - slot).start()

            gather(c, slot).wait()
            for i in range(_BC):
                pltpu.sync_copy(
                    rows_v.at[slot, pl.ds(i * l, l)],
                    out_hbm.at[base_b + c * _BC + i],
                )
            return carry

        lax.fori_loop(0, n_ch, body, 0)

    return k(weight2, x_flat)


def kernel(x, orig_weight, aw1, aw2):
    b, l = x.shape
    v, d = orig_weight.shape
    wpad = _fuse_table(orig_weight, aw1, aw2)
    weight2 = wpad.reshape(2 * v, d)
    return _sc_gather(weight2, x.reshape(-1), b, l, d)
